# baseline (speedup 1.0000x reference)
import functools
import math

import jax
import jax.numpy as jnp
from jax import lax
from jax.experimental import pallas as pl
from jax.experimental.pallas import tpu as pltpu
from jax.experimental.pallas import tpu_sc as plsc


def _sc_gather(table, idx):
    info = plsc.get_sparse_core_info()
    nw = info.num_cores * info.num_subcores
    b, d = idx.shape[0], table.shape[1]
    assert b % nw == 0
    b_per_w = b // nw
    ch = b_per_w
    for cand in (512, 400, 320, 256, 200, 128, 80, 40, 8):
        if b_per_w % cand == 0:
            ch = cand
            break
    nch = b_per_w // ch
    mesh = plsc.VectorSubcoreMesh(core_axis_name="c", subcore_axis_name="s")

    @functools.partial(
        pl.kernel,
        mesh=mesh,
        out_type=jax.ShapeDtypeStruct((b, d), table.dtype),
        scratch_types=[
            pltpu.VMEM((ch,), jnp.int32),
            pltpu.VMEM((ch, d), table.dtype),
            pltpu.SemaphoreType.DMA,
        ],
    )
    def gk(table_hbm, idx_hbm, out_hbm, idx_v, rows_v, sem):
        wid = lax.axis_index("s") * info.num_cores + lax.axis_index("c")

        def body(c, carry):
            base = wid * b_per_w + c * ch
            pltpu.sync_copy(idx_hbm.at[pl.ds(base, ch)], idx_v)
            pltpu.async_copy(table_hbm.at[idx_v], rows_v, sem).wait()
            pltpu.sync_copy(rows_v, out_hbm.at[pl.ds(base, ch)])
            return carry

        lax.fori_loop(0, nch, body, 0)

    return gk(table, idx)


_INV_SQRT2 = 1.0 / math.sqrt(2.0)


def _gelu(x):
    return 0.5 * x * (1.0 + lax.erf(x * _INV_SQRT2))


def _gelu_u(tp):
    return tp * (1.0 + lax.erf(tp))


def _ln(x, g, bb):
    mu = jnp.mean(x, axis=-1, keepdims=True)
    xc = x - mu
    var = jnp.mean(xc * xc, axis=-1, keepdims=True)
    return xc * lax.rsqrt(var + 1e-5) * g + bb


def _decoder_body(num_layers, bk, k, d,
                  wh_r, wc_r, b1_r, w2_r, b2_r, w3s_r, b3e_r,
                  wd1_r, bd1_r, wd2_r, bd2_r, g1_r, n1_r, g2_r, n2_r,
                  e_r, gt_r, x_r, m_r, o_r):
    f32 = jnp.float32
    bf16 = jnp.bfloat16
    eg = jnp.concatenate(
        [e_r[...].astype(bf16), gt_r[...].astype(bf16)], axis=-1)
    h = x_r[...]
    msk = m_r[...]
    r = bk * k
    for l in range(num_layers):
        a = jnp.dot(h.astype(bf16), wh_r[l], preferred_element_type=f32) + b1_r[l]
        t = jnp.dot(eg, wc_r[l], preferred_element_type=f32)
        t = t.reshape(bk, k, d) + a[:, None, :]
        t = _gelu_u(t).reshape(r, d).astype(bf16)
        t = _gelu_u(jnp.dot(t, w2_r[l], preferred_element_type=f32) + b2_r[l])
        s = t.reshape(bk, k, d).sum(axis=1)
        agg = jnp.dot(s, w3s_r[l], preferred_element_type=f32) + b3e_r[l]
        h = _ln(h + agg, g1_r[l], n1_r[l])
        hd = _gelu_u(jnp.dot(h.astype(bf16), wd1_r[l], preferred_element_type=f32) + bd1_r[l]).astype(bf16)
        hd = jnp.dot(hd, wd2_r[l], preferred_element_type=f32) + bd2_r[l]
        h = _ln(h + hd, g2_r[l], n2_r[l])
        h = h * msk
    o_r[...] = h


def _pick_block(n):
    for cand in (400, 320, 256, 200, 160, 128, 80, 64, 40, 16, 8):
        if n % cand == 0:
            return cand
    return n


def _decoder_tc(e2, g2, x0, mask2, packed, nc, off_blocks):
    n, d = x0.shape
    r_total = e2.shape[0]
    k = r_total // n
    num_layers = packed[0].shape[0]
    bk = _pick_block(nc)
    grid = (nc // bk,)

    w_specs = [pl.BlockSpec(w.shape, lambda i: (0,) * 3) for w in packed]
    in_specs = w_specs + [
        pl.BlockSpec((bk * k, d), lambda i: (i + off_blocks, 0)),
        pl.BlockSpec((bk * k, d), lambda i: (i, 0)),
        pl.BlockSpec((bk, d), lambda i: (i + off_blocks, 0)),
        pl.BlockSpec((bk, 1), lambda i: (i + off_blocks, 0)),
    ]
    body = functools.partial(_decoder_body, num_layers, bk, k, d)
    return pl.pallas_call(
        body,
        grid=grid,
        in_specs=in_specs,
        out_specs=pl.BlockSpec((bk, d), lambda i: (i, 0)),
        out_shape=jax.ShapeDtypeStruct((nc, d), jnp.float32),
    )(*packed, e2, g2, x0, mask2)


def _pack_params(params, d, k):
    bf16 = jnp.bfloat16
    c = 1.0 / math.sqrt(2.0)
    l3 = lambda f: jnp.stack([f(p) for p in params])
    wh = l3(lambda p: p["message"][0]["W"][:, 0:d].T * c).astype(bf16)
    wc = l3(lambda p: jnp.concatenate(
        [p["message"][0]["W"][:, d:2 * d].T,
         p["message"][0]["W"][:, 3 * d:4 * d].T], axis=0) * c).astype(bf16)
    b1 = l3(lambda p: p["message"][0]["b"][None, :] * c)
    w2 = l3(lambda p: p["message"][1]["W"].T * (c * c)).astype(bf16)
    b2 = l3(lambda p: p["message"][1]["b"][None, :] * c)
    w3s = l3(lambda p: p["message"][2]["W"].T) * (c / 30.0)
    b3e = l3(lambda p: p["message"][2]["b"][None, :]) * (k / 30.0)
    wd1 = l3(lambda p: p["dense"][0]["W"].T * c).astype(bf16)
    bd1 = l3(lambda p: p["dense"][0]["b"][None, :] * c)
    wd2 = l3(lambda p: p["dense"][1]["W"].T * c).astype(bf16)
    bd2 = l3(lambda p: p["dense"][1]["b"][None, :])
    g1 = l3(lambda p: p["norm1"]["g"][None, :])
    n1 = l3(lambda p: p["norm1"]["b"][None, :])
    g2 = l3(lambda p: p["norm2"]["g"][None, :])
    n2 = l3(lambda p: p["norm2"]["b"][None, :])
    return (wh, wc, b1, w2, b2, w3s, b3e,
            wd1, bd1, wd2, bd2, g1, n1, g2, n2)


def kernel(node_features, edge_features, neighbor_indices, mask, params):
    n, d = node_features.shape
    k = neighbor_indices.shape[1]
    idx = neighbor_indices.astype(jnp.int32).reshape(-1)
    e2 = edge_features.reshape(n * k, d)
    mask2 = mask.astype(jnp.float32).reshape(n, 1)
    packed = _pack_params(params, d, k)

    nc = n // 2
    if n % 2 == 0 and (nc * k) % 256 == 0 and nc % _pick_block(nc) == 0:
        nb = nc // _pick_block(nc)
        g0 = _sc_gather(node_features, idx[: nc * k])
        g1 = _sc_gather(node_features, idx[nc * k:])
        h0 = _decoder_tc(e2, g0, node_features, mask2, packed, nc, 0)
        h1 = _decoder_tc(e2, g1, node_features, mask2, packed, nc, nb)
        return jnp.concatenate([h0, h1], axis=0)

    g2 = _sc_gather(node_features, idx)
    return _decoder_tc(e2, g2, node_features, mask2, packed, n, 0)


# speedup vs baseline: 1.1022x; 1.1022x over previous
import functools
import math

import jax
import jax.numpy as jnp
from jax import lax
from jax.experimental import pallas as pl
from jax.experimental.pallas import tpu as pltpu
from jax.experimental.pallas import tpu_sc as plsc


def _sc_gather(table, idx):
    info = plsc.get_sparse_core_info()
    nw = info.num_cores * info.num_subcores
    b, d = idx.shape[0], table.shape[1]
    assert b % nw == 0
    b_per_w = b // nw
    ch = b_per_w
    for cand in (512, 400, 320, 256, 200, 128, 80, 40, 8):
        if b_per_w % cand == 0:
            ch = cand
            break
    nch = b_per_w // ch
    mesh = plsc.VectorSubcoreMesh(core_axis_name="c", subcore_axis_name="s")

    @functools.partial(
        pl.kernel,
        mesh=mesh,
        out_type=jax.ShapeDtypeStruct((b, d), table.dtype),
        scratch_types=[
            pltpu.VMEM((b_per_w,), jnp.int32),
            pltpu.VMEM((ch, d), table.dtype),
            pltpu.VMEM((ch, d), table.dtype),
            pltpu.SemaphoreType.DMA,
            pltpu.SemaphoreType.DMA,
            pltpu.SemaphoreType.DMA,
            pltpu.SemaphoreType.DMA,
        ],
    )
    def gk(table_hbm, idx_hbm, out_hbm, idx_v, rows_a, rows_b, sga, sgb, swa, swb):
        wid = lax.axis_index("s") * info.num_cores + lax.axis_index("c")
        base = wid * b_per_w
        pltpu.sync_copy(idx_hbm.at[pl.ds(base, b_per_w)], idx_v)

        def pair(p, carry):
            o0 = p * 2 * ch
            o1 = o0 + ch
            ga = pltpu.async_copy(
                table_hbm.at[idx_v.at[pl.ds(o0, ch)]], rows_a, sga)
            gb = pltpu.async_copy(
                table_hbm.at[idx_v.at[pl.ds(o1, ch)]], rows_b, sgb)
            ga.wait()
            wa = pltpu.async_copy(rows_a, out_hbm.at[pl.ds(base + o0, ch)], swa)
            gb.wait()
            wb = pltpu.async_copy(rows_b, out_hbm.at[pl.ds(base + o1, ch)], swb)
            wa.wait()
            wb.wait()
            return carry

        lax.fori_loop(0, nch // 2, pair, 0)

    return gk(table, idx)


_INV_SQRT2 = 1.0 / math.sqrt(2.0)


def _gelu(x):
    return 0.5 * x * (1.0 + lax.erf(x * _INV_SQRT2))


def _gelu_u(tp):
    return tp * (1.0 + lax.erf(tp))


def _ln(x, g, bb):
    mu = jnp.mean(x, axis=-1, keepdims=True)
    xc = x - mu
    var = jnp.mean(xc * xc, axis=-1, keepdims=True)
    return xc * lax.rsqrt(var + 1e-5) * g + bb


def _decoder_body(num_layers, bk, k, d,
                  wh_r, wc_r, b1_r, w2_r, b2_r, w3s_r, b3e_r,
                  wd1_r, bd1_r, wd2_r, bd2_r, g1_r, n1_r, g2_r, n2_r,
                  e_r, gt_r, x_r, m_r, o_r):
    f32 = jnp.float32
    bf16 = jnp.bfloat16
    eg = jnp.concatenate(
        [e_r[...].astype(bf16), gt_r[...].astype(bf16)], axis=-1)
    h = x_r[...]
    msk = m_r[...]
    r = bk * k
    for l in range(num_layers):
        a = jnp.dot(h.astype(bf16), wh_r[l], preferred_element_type=f32) + b1_r[l]
        t = jnp.dot(eg, wc_r[l], preferred_element_type=f32)
        t = t.reshape(bk, k, d) + a[:, None, :]
        t = _gelu_u(t).reshape(r, d).astype(bf16)
        t = _gelu_u(jnp.dot(t, w2_r[l], preferred_element_type=f32) + b2_r[l])
        s = t.reshape(bk, k, d).sum(axis=1)
        agg = jnp.dot(s, w3s_r[l], preferred_element_type=f32) + b3e_r[l]
        h = _ln(h + agg, g1_r[l], n1_r[l])
        hd = _gelu_u(jnp.dot(h.astype(bf16), wd1_r[l], preferred_element_type=f32) + bd1_r[l]).astype(bf16)
        hd = jnp.dot(hd, wd2_r[l], preferred_element_type=f32) + bd2_r[l]
        h = _ln(h + hd, g2_r[l], n2_r[l])
        h = h * msk
    o_r[...] = h


def _pick_block(n):
    for cand in (400, 320, 256, 200, 160, 128, 80, 64, 40, 16, 8):
        if n % cand == 0:
            return cand
    return n


def _decoder_tc(e2, g2, x0, mask2, packed, nc, off_blocks):
    n, d = x0.shape
    r_total = e2.shape[0]
    k = r_total // n
    num_layers = packed[0].shape[0]
    bk = _pick_block(nc)
    grid = (nc // bk,)

    w_specs = [pl.BlockSpec(w.shape, lambda i: (0,) * 3) for w in packed]
    in_specs = w_specs + [
        pl.BlockSpec((bk * k, d), lambda i: (i + off_blocks, 0)),
        pl.BlockSpec((bk * k, d), lambda i: (i, 0)),
        pl.BlockSpec((bk, d), lambda i: (i + off_blocks, 0)),
        pl.BlockSpec((bk, 1), lambda i: (i + off_blocks, 0)),
    ]
    body = functools.partial(_decoder_body, num_layers, bk, k, d)
    return pl.pallas_call(
        body,
        grid=grid,
        in_specs=in_specs,
        out_specs=pl.BlockSpec((bk, d), lambda i: (i, 0)),
        out_shape=jax.ShapeDtypeStruct((nc, d), jnp.float32),
    )(*packed, e2, g2, x0, mask2)


def _pack_params(params, d, k):
    bf16 = jnp.bfloat16
    c = 1.0 / math.sqrt(2.0)
    l3 = lambda f: jnp.stack([f(p) for p in params])
    wh = l3(lambda p: p["message"][0]["W"][:, 0:d].T * c).astype(bf16)
    wc = l3(lambda p: jnp.concatenate(
        [p["message"][0]["W"][:, d:2 * d].T,
         p["message"][0]["W"][:, 3 * d:4 * d].T], axis=0) * c).astype(bf16)
    b1 = l3(lambda p: p["message"][0]["b"][None, :] * c)
    w2 = l3(lambda p: p["message"][1]["W"].T * (c * c)).astype(bf16)
    b2 = l3(lambda p: p["message"][1]["b"][None, :] * c)
    w3s = l3(lambda p: p["message"][2]["W"].T) * (c / 30.0)
    b3e = l3(lambda p: p["message"][2]["b"][None, :]) * (k / 30.0)
    wd1 = l3(lambda p: p["dense"][0]["W"].T * c).astype(bf16)
    bd1 = l3(lambda p: p["dense"][0]["b"][None, :] * c)
    wd2 = l3(lambda p: p["dense"][1]["W"].T * c).astype(bf16)
    bd2 = l3(lambda p: p["dense"][1]["b"][None, :])
    g1 = l3(lambda p: p["norm1"]["g"][None, :])
    n1 = l3(lambda p: p["norm1"]["b"][None, :])
    g2 = l3(lambda p: p["norm2"]["g"][None, :])
    n2 = l3(lambda p: p["norm2"]["b"][None, :])
    return (wh, wc, b1, w2, b2, w3s, b3e,
            wd1, bd1, wd2, bd2, g1, n1, g2, n2)


def kernel(node_features, edge_features, neighbor_indices, mask, params):
    n, d = node_features.shape
    k = neighbor_indices.shape[1]
    idx = neighbor_indices.astype(jnp.int32).reshape(-1)
    e2 = edge_features.reshape(n * k, d)
    mask2 = mask.astype(jnp.float32).reshape(n, 1)
    packed = _pack_params(params, d, k)
    g2 = _sc_gather(node_features, idx)
    return _decoder_tc(e2, g2, node_features, mask2, packed, n, 0)
